# baseline (speedup 1.0000x reference)
import jax
import jax.numpy as jnp
from jax import lax
from jax.experimental import pallas as pl
from jax.experimental.pallas import tpu as pltpu
from jax.experimental.pallas import tpu_sc as plsc

N = 10000
E = 320000
C = 128
ALPHA1 = 0.9
ALPHA2 = 0.7
NPROP1 = 10
NPROP2 = 10

NS = 16
NC = 2
CH = 128
NH = N // NC
RT = 313
RC = 64
NRC = 5
N_PAD = N + 16
A_PAD = NH + 24
DUMMY = NH + 8
NG = C // 16
NTR = 5000

ROWS_BLK = 400


def _softmax_body(x_ref, o_ref):
    x = x_ref[...]
    m = jnp.max(x, axis=-1, keepdims=True)
    e = jnp.exp(x - m)
    o_ref[...] = e / jnp.sum(e, axis=-1, keepdims=True)


def _softmax_tc(x):
    return pl.pallas_call(
        _softmax_body,
        grid=(N // ROWS_BLK,),
        in_specs=[pl.BlockSpec((ROWS_BLK, C), lambda i: (i, 0))],
        out_specs=pl.BlockSpec((ROWS_BLK, C), lambda i: (i, 0)),
        out_shape=jax.ShapeDtypeStruct((N, C), jnp.float32),
    )(x)


def _sc_body(p_h, y_h, tr_h, srcs_h, dstl_h, eb_h,
             out_h, res_h, x0_h,
             bvm, isb, idb, msb, mdb, g0, g1, abuf, xbuf, dsl, msl, trbuf,
             acc, rs0, rs1, rd0, rd1, gg0, gg1):
    c = lax.axis_index("c")
    s = lax.axis_index("s")
    w = c * NS + s
    lo_l = s * RT
    nrows = jnp.minimum(RT, NH - lo_l)
    grow0 = c * NH + lo_l

    f32 = jnp.float32
    i32 = jnp.int32
    ones16 = jnp.ones((16,), f32)
    zero16 = jnp.zeros((16,), f32)
    iota0 = jnp.zeros((16,), i32)
    lane = lax.iota(i32, 16)
    lane0 = lane == 0

    pltpu.sync_copy(eb_h, bvm)
    pltpu.sync_copy(tr_h, trbuf)

    def sca(ref, i):
        return plsc.load_gather(ref, [iota0 + i])[0]

    lo_e = sca(bvm, w)
    hi_e = sca(bvm, w + 1)
    base0 = (lo_e // 8) * 8
    nch = (hi_e - base0 + (CH - 1)) // CH
    npair = nch // 2

    def fill16(ref, nrow, ncolg, val):
        def fb(i, _):
            for g in range(ncolg):
                ref[i, pl.ds(g * 16, 16)] = val
            return 0
        lax.fori_loop(0, nrow, fb, 0)

    def fetch_raw(e, sl):
        off = base0 + e * CH
        pltpu.async_copy(srcs_h.at[pl.ds(off, CH)], isb.at[sl], (rs0, rs1)[sl])
        pltpu.async_copy(dstl_h.at[pl.ds(off, CH)], idb.at[sl], (rd0, rd1)[sl])

    def wait_raw(e, sl):
        off = base0 + e * CH
        pltpu.make_async_copy(srcs_h.at[pl.ds(off, CH)], isb.at[sl],
                              (rs0, rs1)[sl]).wait()
        pltpu.make_async_copy(dstl_h.at[pl.ds(off, CH)], idb.at[sl],
                              (rd0, rd1)[sl]).wait()

    def mask_pass(e, sl):
        off = base0 + e * CH
        for g in range(CH // 16):
            gsl = pl.ds(g * 16, 16)
            pos = iota0 + (off + g * 16) + lane
            valid = jnp.logical_and(pos >= lo_e, pos < hi_e)
            msb[sl, gsl] = jnp.where(valid, isb[sl, gsl], N)
            mdb[sl, gsl] = jnp.where(valid, idb[sl, gsl], DUMMY)

    def gather(e, sl):
        pltpu.async_copy(res_h.at[msb.at[sl]], (g0, g1)[sl], (gg0, gg1)[sl])

    def wait_gather(e, sl):
        pltpu.make_async_copy(res_h.at[msb.at[sl]], (g0, g1)[sl],
                              (gg0, gg1)[sl]).wait()

    def scatter(e, sl):
        pltpu.sync_copy((g0, g1)[sl], acc.at[mdb.at[sl]], add=True)

    def edge_sweep(scatter_fn):
        @pl.when(nch > 0)
        def _pro():
            fetch_raw(0, 0)

            @pl.when(nch > 1)
            def _pro1():
                fetch_raw(1, 1)

        def pair_body(j, _):
            e0 = 2 * j
            e1 = e0 + 1
            wait_raw(e0, 0)
            mask_pass(e0, 0)
            gather(e0, 0)
            wait_raw(e1, 1)
            mask_pass(e1, 1)
            gather(e1, 1)

            @pl.when(e0 + 2 < nch)
            def _f0():
                fetch_raw(e0 + 2, 0)

            @pl.when(e1 + 2 < nch)
            def _f1():
                fetch_raw(e1 + 2, 1)
            wait_gather(e0, 0)
            scatter_fn(e0, 0)
            wait_gather(e1, 1)
            scatter_fn(e1, 1)
            return 0
        lax.fori_loop(0, npair, pair_body, 0)

        @pl.when(nch - 2 * npair > 0)
        def _tail():
            e = nch - 1
            wait_raw(e, 0)
            mask_pass(e, 0)
            gather(e, 0)
            wait_gather(e, 0)
            scatter_fn(e, 0)

    def zero_acc():
        fill16(xbuf, RC, NG, zero16)
        for k in range(NRC):
            st = jnp.minimum(k * RC, nrows - RC)
            pltpu.sync_copy(xbuf, acc.at[pl.ds(lo_l + st, RC)])

        @pl.when(s == NS - 1)
        def _zpad():
            pltpu.sync_copy(xbuf.at[pl.ds(0, 24)], acc.at[pl.ds(NH, 24)])

    zero_acc()
    plsc.subcore_barrier()
    fill16(g0, CH, NG, ones16)

    def deg_scatter(e, sl):
        pltpu.sync_copy(g0, acc.at[mdb.at[sl]], add=True)

    def deg_gather(e, sl):
        pass

    @pl.when(nch > 0)
    def _dpro():
        fetch_raw(0, 0)

    def deg_body(j, _):
        wait_raw(j, 0)
        mask_pass(j, 0)

        @pl.when(j + 1 < nch)
        def _dn():
            fetch_raw(j + 1, 0)
        pltpu.sync_copy(g0, acc.at[mdb.at[0]], add=True)
        return 0
    lax.fori_loop(0, nch, deg_body, 0)
    plsc.subcore_barrier()

    magic = jnp.full((16,), 0x5F3759DF, i32)
    for k in range(NRC):
        st = jnp.minimum(k * RC, nrows - RC)
        pltpu.sync_copy(acc.at[pl.ds(lo_l + st, RC)], abuf)

        def newton_body(i, _):
            d = jnp.maximum(abuf[i, pl.ds(0, 16)], 1.0)
            yi = magic - jax.lax.shift_right_logical(plsc.bitcast(d, i32), 1)
            yf = plsc.bitcast(yi, f32)
            half = -0.5 * d
            for _ in range(3):
                yf = yf * (1.5 + half * yf * yf)
            plsc.store_scatter(dsl, [iota0 + (st + i)], yf, mask=lane0)
            return 0
        lax.fori_loop(0, RC, newton_body, 0)

    def fm(i, _):
        msl[pl.ds(i * 16, 16)] = zero16
        return 0
    lax.fori_loop(0, 320 // 16, fm, 0)

    def mask_body(t, _):
        base = jnp.minimum(t * 16, NTR - 16)
        idx = trbuf[pl.ds(base, 16)] - grow0
        inr = jnp.logical_and(idx >= 0, idx < nrows)
        lidx = jnp.where(inr, idx, 0)
        plsc.store_scatter(msl, [lidx], ones16, mask=inr)
        return 0
    lax.fori_loop(0, (NTR + 15) // 16, mask_body, 0)

    for k in range(NRC):
        st = jnp.minimum(k * RC, nrows - RC)
        rows = pl.ds(grow0 + st, RC)
        pltpu.sync_copy(p_h.at[rows], g0.at[pl.ds(0, RC)])
        pltpu.sync_copy(y_h.at[rows], g1.at[pl.ds(0, RC)])

        def p0_body(i, _):
            li = iota0 + (st + i)
            m = plsc.load_gather(msl, [li])
            d = plsc.load_gather(dsl, [li])
            for g in range(NG):
                sl = pl.ds(g * 16, 16)
                e = m * (g1[i, sl] - g0[i, sl])
                abuf[i, sl] = d * e
                xbuf[i, sl] = (1.0 - ALPHA1) * e
            return 0
        lax.fori_loop(0, RC, p0_body, 0)
        pltpu.sync_copy(abuf, res_h.at[rows])
        pltpu.sync_copy(xbuf, x0_h.at[rows])

    plsc.subcore_barrier()

    def prop(alpha, n_iter):
        def it_body(it, _):
            zero_acc()
            plsc.subcore_barrier()
            edge_sweep(scatter)
            plsc.subcore_barrier()

            for k in range(NRC):
                st = jnp.minimum(k * RC, nrows - RC)
                rows = pl.ds(grow0 + st, RC)
                pltpu.sync_copy(acc.at[pl.ds(lo_l + st, RC)], abuf)
                pltpu.sync_copy(x0_h.at[rows], xbuf)

                def nw_body(i, _):
                    d = plsc.load_gather(dsl, [iota0 + (st + i)])
                    da = d * alpha
                    for g in range(NG):
                        sl = pl.ds(g * 16, 16)
                        abuf[i, sl] = d * (da * abuf[i, sl] + xbuf[i, sl])
                    return 0
                lax.fori_loop(0, RC, nw_body, 0)
                pltpu.sync_copy(abuf, res_h.at[rows])
            plsc.subcore_barrier()
            return 0
        lax.fori_loop(0, n_iter, it_body, 0)

    prop(ALPHA1, NPROP1)

    for k in range(NRC):
        st = jnp.minimum(k * RC, nrows - RC)
        rows = pl.ds(grow0 + st, RC)
        pltpu.sync_copy(res_h.at[rows], abuf)
        pltpu.sync_copy(p_h.at[rows], g0.at[pl.ds(0, RC)])
        pltpu.sync_copy(y_h.at[rows], g1.at[pl.ds(0, RC)])

        def tr_body(i, _):
            li = iota0 + (st + i)
            m = plsc.load_gather(msl, [li])
            d = plsc.load_gather(dsl, [li])
            for g in range(NG):
                sl = pl.ds(g * 16, 16)
                err = abuf[i, sl] / d
                corr = g0[i, sl] + err
                h0 = m * g1[i, sl] + (1.0 - m) * corr
                abuf[i, sl] = d * h0
                xbuf[i, sl] = (1.0 - ALPHA2) * h0
            return 0
        lax.fori_loop(0, RC, tr_body, 0)
        pltpu.sync_copy(abuf, res_h.at[rows])
        pltpu.sync_copy(xbuf, x0_h.at[rows])

    plsc.subcore_barrier()

    prop(ALPHA2, NPROP2)

    for k in range(NRC):
        st = jnp.minimum(k * RC, nrows - RC)
        rows = pl.ds(grow0 + st, RC)
        pltpu.sync_copy(res_h.at[rows], abuf)

        def out_body(i, _):
            d = plsc.load_gather(dsl, [iota0 + (st + i)])
            for g in range(NG):
                sl = pl.ds(g * 16, 16)
                abuf[i, sl] = abuf[i, sl] / d
            return 0
        lax.fori_loop(0, RC, out_body, 0)
        pltpu.sync_copy(abuf, out_h.at[rows])


@jax.jit
def _sc_call(p, y, train_idx, srcs, dstl, eb):
    mesh = plsc.VectorSubcoreMesh(core_axis_name="c", subcore_axis_name="s")
    f = pl.kernel(
        _sc_body,
        out_type=[
            jax.ShapeDtypeStruct((N, C), jnp.float32),
            jax.ShapeDtypeStruct((N_PAD, C), jnp.float32),
            jax.ShapeDtypeStruct((N, C), jnp.float32),
        ],
        mesh=mesh,
        compiler_params=pltpu.CompilerParams(use_tc_tiling_on_sc=False,
                                             needs_layout_passes=False),
        scratch_types=[
            pltpu.VMEM((40,), jnp.int32),
            pltpu.VMEM((2, CH), jnp.int32),
            pltpu.VMEM((2, CH), jnp.int32),
            pltpu.VMEM((2, CH), jnp.int32),
            pltpu.VMEM((2, CH), jnp.int32),
            pltpu.VMEM((CH, C), jnp.float32),
            pltpu.VMEM((CH, C), jnp.float32),
            pltpu.VMEM((RC, C), jnp.float32),
            pltpu.VMEM((RC, C), jnp.float32),
            pltpu.VMEM((320,), jnp.float32),
            pltpu.VMEM((320,), jnp.float32),
            pltpu.VMEM((NTR,), jnp.int32),
            pltpu.VMEM_SHARED((A_PAD, C), jnp.float32),
            pltpu.SemaphoreType.DMA,
            pltpu.SemaphoreType.DMA,
            pltpu.SemaphoreType.DMA,
            pltpu.SemaphoreType.DMA,
            pltpu.SemaphoreType.DMA,
            pltpu.SemaphoreType.DMA,
        ],
    )
    return f(p, y, train_idx, srcs, dstl, eb)


def kernel(model_out, edge_index, y, train_idx):
    p = _softmax_tc(model_out)
    src = edge_index[0]
    dst = edge_index[1]
    order = jnp.argsort(dst)
    srcs = src[order]
    dsts = dst[order]
    dstl = dsts - NH * (dsts // NH)
    srcs = jnp.concatenate([srcs, jnp.full((2 * CH,), N, jnp.int32)])
    dstl = jnp.concatenate([dstl, jnp.full((2 * CH,), DUMMY, jnp.int32)])
    bounds = []
    for cc in range(NC):
        for ss in range(NS):
            bounds.append(cc * NH + min(ss * RT, NH))
    bounds.append(N)
    eb = jnp.searchsorted(dsts, jnp.array(bounds, jnp.int32)).astype(jnp.int32)
    eb = jnp.concatenate([eb, jnp.zeros((7,), jnp.int32)])
    out, _, _ = _sc_call(p, y, train_idx, srcs, dstl, eb)
    return out


# speedup vs baseline: 1.1734x; 1.1734x over previous
import jax
import jax.numpy as jnp
from jax import lax
from jax.experimental import pallas as pl
from jax.experimental.pallas import tpu as pltpu
from jax.experimental.pallas import tpu_sc as plsc

N = 10000
E = 320000
C = 128
ALPHA1 = 0.9
ALPHA2 = 0.7
NPROP1 = 10
NPROP2 = 10

NS = 16
NC = 2
CH = 128
NH = N // NC
RT = 313
RC = 64
NRC = 5
N_PAD = N + 16
A_PAD = NH + 24
DUMMY = NH + 8
NG = C // 16
NTR = 5000

ROWS_BLK = 400


def _softmax_body(x_ref, o_ref):
    x = x_ref[...]
    m = jnp.max(x, axis=-1, keepdims=True)
    e = jnp.exp(x - m)
    o_ref[...] = e / jnp.sum(e, axis=-1, keepdims=True)


def _softmax_tc(x):
    return pl.pallas_call(
        _softmax_body,
        grid=(N // ROWS_BLK,),
        in_specs=[pl.BlockSpec((ROWS_BLK, C), lambda i: (i, 0))],
        out_specs=pl.BlockSpec((ROWS_BLK, C), lambda i: (i, 0)),
        out_shape=jax.ShapeDtypeStruct((N, C), jnp.float32),
    )(x)


def _sc_body(p_h, y_h, tr_h, srcs_h, dstl_h, eb_h,
             out_h, res_h, x0_h,
             bvm, isb, idb, msb, mdb, g0, g1, abuf, xbuf, dsl, msl, trbuf,
             acc, rs0, rs1, rd0, rd1, gg0, gg1):
    c = lax.axis_index("c")
    s = lax.axis_index("s")
    w = c * NS + s
    lo_l = s * RT
    nrows = jnp.minimum(RT, NH - lo_l)
    grow0 = c * NH + lo_l

    f32 = jnp.float32
    i32 = jnp.int32
    ones16 = jnp.ones((16,), f32)
    zero16 = jnp.zeros((16,), f32)
    iota0 = jnp.zeros((16,), i32)
    lane = lax.iota(i32, 16)
    lane0 = lane == 0

    pltpu.sync_copy(eb_h, bvm)
    pltpu.sync_copy(tr_h, trbuf)

    def sca(ref, i):
        return plsc.load_gather(ref, [iota0 + i])[0]

    lo_e = sca(bvm, w)
    hi_e = sca(bvm, w + 1)
    base0 = (lo_e // 8) * 8
    nch = (hi_e - base0 + (CH - 1)) // CH
    npair = nch // 2

    def fill16(ref, nrow, ncolg, val):
        def fb(i, _):
            for g in range(ncolg):
                ref[i, pl.ds(g * 16, 16)] = val
            return 0
        lax.fori_loop(0, nrow, fb, 0)

    def fetch_raw(e, sl):
        off = base0 + e * CH
        pltpu.async_copy(srcs_h.at[pl.ds(off, CH)], isb.at[sl], (rs0, rs1)[sl])
        pltpu.async_copy(dstl_h.at[pl.ds(off, CH)], idb.at[sl], (rd0, rd1)[sl])

    def wait_raw(e, sl):
        off = base0 + e * CH
        pltpu.make_async_copy(srcs_h.at[pl.ds(off, CH)], isb.at[sl],
                              (rs0, rs1)[sl]).wait()
        pltpu.make_async_copy(dstl_h.at[pl.ds(off, CH)], idb.at[sl],
                              (rd0, rd1)[sl]).wait()

    def mask_pass(e, sl):
        off = base0 + e * CH
        for g in range(CH // 16):
            gsl = pl.ds(g * 16, 16)
            pos = iota0 + (off + g * 16) + lane
            valid = jnp.logical_and(pos >= lo_e, pos < hi_e)
            msb[sl, gsl] = jnp.where(valid, isb[sl, gsl], N)
            mdb[sl, gsl] = jnp.where(valid, idb[sl, gsl], DUMMY)

    def gather(e, sl):
        pltpu.async_copy(res_h.at[msb.at[sl]], (g0, g1)[sl], (gg0, gg1)[sl])

    def wait_gather(e, sl):
        pltpu.make_async_copy(res_h.at[msb.at[sl]], (g0, g1)[sl],
                              (gg0, gg1)[sl]).wait()

    def scatter(e, sl):
        pltpu.sync_copy((g0, g1)[sl], acc.at[mdb.at[sl]], add=True)

    def edge_sweep(scatter_fn):
        @pl.when(nch > 0)
        def _pro():
            fetch_raw(0, 0)
            wait_raw(0, 0)
            mask_pass(0, 0)
            gather(0, 0)

            @pl.when(nch > 1)
            def _pro1():
                fetch_raw(1, 1)

        def pair_body(j, _):
            e0 = 2 * j
            e1 = e0 + 1
            wait_raw(e1, 1)
            mask_pass(e1, 1)
            gather(e1, 1)

            @pl.when(e0 + 2 < nch)
            def _f0():
                fetch_raw(e0 + 2, 0)
            wait_gather(e0, 0)
            scatter_fn(e0, 0)

            @pl.when(e0 + 2 < nch)
            def _g0():
                wait_raw(e0 + 2, 0)
                mask_pass(e0 + 2, 0)
                gather(e0 + 2, 0)

            @pl.when(e1 + 2 < nch)
            def _f1():
                fetch_raw(e1 + 2, 1)
            wait_gather(e1, 1)
            scatter_fn(e1, 1)
            return 0
        lax.fori_loop(0, npair, pair_body, 0)

        @pl.when(nch - 2 * npair > 0)
        def _tail():
            e = nch - 1
            wait_gather(e, 0)
            scatter_fn(e, 0)

    def zero_acc():
        fill16(xbuf, RC, NG, zero16)
        for k in range(NRC):
            st = jnp.minimum(k * RC, nrows - RC)
            pltpu.sync_copy(xbuf, acc.at[pl.ds(lo_l + st, RC)])

        @pl.when(s == NS - 1)
        def _zpad():
            pltpu.sync_copy(xbuf.at[pl.ds(0, 24)], acc.at[pl.ds(NH, 24)])

    zero_acc()
    plsc.subcore_barrier()
    fill16(g0, CH, NG, ones16)

    def deg_scatter(e, sl):
        pltpu.sync_copy(g0, acc.at[mdb.at[sl]], add=True)

    def deg_gather(e, sl):
        pass

    @pl.when(nch > 0)
    def _dpro():
        fetch_raw(0, 0)

    def deg_body(j, _):
        wait_raw(j, 0)
        mask_pass(j, 0)

        @pl.when(j + 1 < nch)
        def _dn():
            fetch_raw(j + 1, 0)
        pltpu.sync_copy(g0, acc.at[mdb.at[0]], add=True)
        return 0
    lax.fori_loop(0, nch, deg_body, 0)
    plsc.subcore_barrier()

    magic = jnp.full((16,), 0x5F3759DF, i32)
    for k in range(NRC):
        st = jnp.minimum(k * RC, nrows - RC)
        pltpu.sync_copy(acc.at[pl.ds(lo_l + st, RC)], abuf)

        def newton_body(i, _):
            d = jnp.maximum(abuf[i, pl.ds(0, 16)], 1.0)
            yi = magic - jax.lax.shift_right_logical(plsc.bitcast(d, i32), 1)
            yf = plsc.bitcast(yi, f32)
            half = -0.5 * d
            for _ in range(3):
                yf = yf * (1.5 + half * yf * yf)
            plsc.store_scatter(dsl, [iota0 + (st + i)], yf, mask=lane0)
            return 0
        lax.fori_loop(0, RC, newton_body, 0)

    def fm(i, _):
        msl[pl.ds(i * 16, 16)] = zero16
        return 0
    lax.fori_loop(0, 320 // 16, fm, 0)

    def mask_body(t, _):
        base = jnp.minimum(t * 16, NTR - 16)
        idx = trbuf[pl.ds(base, 16)] - grow0
        inr = jnp.logical_and(idx >= 0, idx < nrows)
        lidx = jnp.where(inr, idx, 0)
        plsc.store_scatter(msl, [lidx], ones16, mask=inr)
        return 0
    lax.fori_loop(0, (NTR + 15) // 16, mask_body, 0)

    for k in range(NRC):
        st = jnp.minimum(k * RC, nrows - RC)
        rows = pl.ds(grow0 + st, RC)
        pltpu.sync_copy(p_h.at[rows], g0.at[pl.ds(0, RC)])
        pltpu.sync_copy(y_h.at[rows], g1.at[pl.ds(0, RC)])

        def p0_body(i, _):
            li = iota0 + (st + i)
            m = plsc.load_gather(msl, [li])
            d = plsc.load_gather(dsl, [li])
            for g in range(NG):
                sl = pl.ds(g * 16, 16)
                e = m * (g1[i, sl] - g0[i, sl])
                abuf[i, sl] = d * e
                xbuf[i, sl] = (1.0 - ALPHA1) * e
            return 0
        lax.fori_loop(0, RC, p0_body, 0)
        pltpu.sync_copy(abuf, res_h.at[rows])
        pltpu.sync_copy(xbuf, x0_h.at[rows])

    plsc.subcore_barrier()

    def prop(alpha, n_iter):
        def it_body(it, _):
            zero_acc()
            plsc.subcore_barrier()
            edge_sweep(scatter)
            plsc.subcore_barrier()

            for k in range(NRC):
                st = jnp.minimum(k * RC, nrows - RC)
                rows = pl.ds(grow0 + st, RC)
                pltpu.sync_copy(acc.at[pl.ds(lo_l + st, RC)], abuf)
                pltpu.sync_copy(x0_h.at[rows], xbuf)

                def nw_body(i, _):
                    d = plsc.load_gather(dsl, [iota0 + (st + i)])
                    da = d * alpha
                    for g in range(NG):
                        sl = pl.ds(g * 16, 16)
                        abuf[i, sl] = d * (da * abuf[i, sl] + xbuf[i, sl])
                    return 0
                lax.fori_loop(0, RC, nw_body, 0)
                pltpu.sync_copy(abuf, res_h.at[rows])
            plsc.subcore_barrier()
            return 0
        lax.fori_loop(0, n_iter, it_body, 0)

    prop(ALPHA1, NPROP1)

    for k in range(NRC):
        st = jnp.minimum(k * RC, nrows - RC)
        rows = pl.ds(grow0 + st, RC)
        pltpu.sync_copy(res_h.at[rows], abuf)
        pltpu.sync_copy(p_h.at[rows], g0.at[pl.ds(0, RC)])
        pltpu.sync_copy(y_h.at[rows], g1.at[pl.ds(0, RC)])

        def tr_body(i, _):
            li = iota0 + (st + i)
            m = plsc.load_gather(msl, [li])
            d = plsc.load_gather(dsl, [li])
            for g in range(NG):
                sl = pl.ds(g * 16, 16)
                err = abuf[i, sl] / d
                corr = g0[i, sl] + err
                h0 = m * g1[i, sl] + (1.0 - m) * corr
                abuf[i, sl] = d * h0
                xbuf[i, sl] = (1.0 - ALPHA2) * h0
            return 0
        lax.fori_loop(0, RC, tr_body, 0)
        pltpu.sync_copy(abuf, res_h.at[rows])
        pltpu.sync_copy(xbuf, x0_h.at[rows])

    plsc.subcore_barrier()

    prop(ALPHA2, NPROP2)

    for k in range(NRC):
        st = jnp.minimum(k * RC, nrows - RC)
        rows = pl.ds(grow0 + st, RC)
        pltpu.sync_copy(res_h.at[rows], abuf)

        def out_body(i, _):
            d = plsc.load_gather(dsl, [iota0 + (st + i)])
            for g in range(NG):
                sl = pl.ds(g * 16, 16)
                abuf[i, sl] = abuf[i, sl] / d
            return 0
        lax.fori_loop(0, RC, out_body, 0)
        pltpu.sync_copy(abuf, out_h.at[rows])


@jax.jit
def _sc_call(p, y, train_idx, srcs, dstl, eb):
    mesh = plsc.VectorSubcoreMesh(core_axis_name="c", subcore_axis_name="s")
    f = pl.kernel(
        _sc_body,
        out_type=[
            jax.ShapeDtypeStruct((N, C), jnp.float32),
            jax.ShapeDtypeStruct((N_PAD, C), jnp.float32),
            jax.ShapeDtypeStruct((N, C), jnp.float32),
        ],
        mesh=mesh,
        compiler_params=pltpu.CompilerParams(use_tc_tiling_on_sc=False,
                                             needs_layout_passes=False),
        scratch_types=[
            pltpu.VMEM((40,), jnp.int32),
            pltpu.VMEM((2, CH), jnp.int32),
            pltpu.VMEM((2, CH), jnp.int32),
            pltpu.VMEM((2, CH), jnp.int32),
            pltpu.VMEM((2, CH), jnp.int32),
            pltpu.VMEM((CH, C), jnp.float32),
            pltpu.VMEM((CH, C), jnp.float32),
            pltpu.VMEM((RC, C), jnp.float32),
            pltpu.VMEM((RC, C), jnp.float32),
            pltpu.VMEM((320,), jnp.float32),
            pltpu.VMEM((320,), jnp.float32),
            pltpu.VMEM((NTR,), jnp.int32),
            pltpu.VMEM_SHARED((A_PAD, C), jnp.float32),
            pltpu.SemaphoreType.DMA,
            pltpu.SemaphoreType.DMA,
            pltpu.SemaphoreType.DMA,
            pltpu.SemaphoreType.DMA,
            pltpu.SemaphoreType.DMA,
            pltpu.SemaphoreType.DMA,
        ],
    )
    return f(p, y, train_idx, srcs, dstl, eb)


def kernel(model_out, edge_index, y, train_idx):
    p = _softmax_tc(model_out)
    src = edge_index[0]
    dst = edge_index[1]
    order = jnp.argsort(dst)
    srcs = src[order]
    dsts = dst[order]
    dstl = dsts - NH * (dsts // NH)
    srcs = jnp.concatenate([srcs, jnp.full((2 * CH,), N, jnp.int32)])
    dstl = jnp.concatenate([dstl, jnp.full((2 * CH,), DUMMY, jnp.int32)])
    bounds = []
    for cc in range(NC):
        for ss in range(NS):
            bounds.append(cc * NH + min(ss * RT, NH))
    bounds.append(N)
    eb = jnp.searchsorted(dsts, jnp.array(bounds, jnp.int32)).astype(jnp.int32)
    eb = jnp.concatenate([eb, jnp.zeros((7,), jnp.int32)])
    out, _, _ = _sc_call(p, y, train_idx, srcs, dstl, eb)
    return out
